# baseline (speedup 1.0000x reference)
import numpy as np
import jax
import jax.numpy as jnp
from jax.experimental import pallas as pl
from jax.experimental.pallas import tpu as pltpu

_V = 25
_C = 60
_KG = 13
_KD = 6
_N = 64
_T = 300
_F32 = jnp.float32

_EDGE_LIST = [(1, 2), (2, 21), (3, 21), (4, 3), (5, 21), (6, 5), (7, 6),
              (8, 7), (9, 21), (10, 9), (11, 10), (12, 11), (13, 1),
              (14, 13), (15, 14), (16, 15), (17, 1), (18, 17), (19, 18),
              (20, 19), (22, 23), (23, 8), (24, 25), (25, 12)]


def _adj_bin():
    A = np.zeros((_V, _V), dtype=np.float64)
    for i, j in _EDGE_LIST:
        A[i - 1, j - 1] = 1.0
        A[j - 1, i - 1] = 1.0
    return A


def _k_adj(A, k):
    n = A.shape[0]
    I = np.eye(n)
    if k == 0:
        return I
    Ak = ((np.linalg.matrix_power(A + I, k) >= 1).astype(np.float64)
          - (np.linalg.matrix_power(A + I, k - 1) >= 1).astype(np.float64))
    return Ak + I


def _norm_adj(A):
    d = A.sum(-1)
    dinv = np.where(d > 0, 1.0 / d, 0.0)
    return A * dinv[:, None]


def _a_pow(A, K):
    return np.concatenate([_norm_adj(_k_adj(A, k)) for k in range(K)], axis=0)


_A1S = _a_pow(_adj_bin(), _KG).reshape(_KG, _V, _V).astype(np.float32)


def _g3d_struct(window):
    A = _adj_bin()
    B = ((A + np.eye(_V)) > 0).astype(np.float64)
    Dall = np.zeros((_V, _KD * _V), dtype=np.float64)
    beta = np.zeros((_KD, _V), dtype=np.float64)
    beta[0, :] = 1.0
    reach_prev = np.eye(_V) > 0
    Bp = np.eye(_V)
    for k in range(1, _KD):
        Bp = Bp @ B
        reach = Bp >= 1
        if k == 1:
            Dk = reach.astype(np.float64)
            ak = 1.0 / (window * Dk.sum(axis=1))
        else:
            Dk = (reach & ~reach_prev).astype(np.float64)
            ak = 1.0 / (window * Dk.sum(axis=1) + 1.0)
            beta[k, :] = ak
        reach_prev = reach
        Dall[:, k * _V:(k + 1) * _V] = (Dk * ak[:, None]).T
    return Dall[:, _V:].astype(np.float32), beta.astype(np.float32)


_D3, _BETA3 = _g3d_struct(3)
_D5, _BETA5 = _g3d_struct(5)


def _dg(a, b, dims):
    return jax.lax.dot_general(a, b, (dims, ((), ())),
                               preferred_element_type=_F32)


_BF16 = jnp.bfloat16


def _dgb(a, b, dims):
    return jax.lax.dot_general(a.astype(_BF16), b.astype(_BF16),
                               (dims, ((), ())),
                               preferred_element_type=_F32)


def _dotb(a, b):
    return jnp.dot(a.astype(_BF16), b.astype(_BF16),
                   preferred_element_type=_F32)


def _gcn1_body(x_ref, m_ref, o_ref):
    o_ref[0] = jnp.maximum(_dotb(x_ref[0], m_ref[...]), 0.0)


def _tcn_a_body(x_ref, w1_ref, w2_ref, wr_ref, o_ref):
    x3 = x_ref[0]
    x2 = x3.reshape(_T * _V, _C)
    y = _dgb(x2, w1_ref[...], ((1,), (1,)))
    y3 = y.reshape(_T, _V, _C)
    yr = jnp.maximum(y3[:, :, :50], 0.0)
    yp = jnp.pad(yr[:, :, :40], ((4, 4), (0, 0), (0, 0)))
    taps = []
    for j in range(3):
        for i in range(4):
            d = i + 1
            off = 4 + (j - 1) * d
            sl = yp[off:off + _T, :, 10 * i:10 * i + 10]
            taps.append(sl.reshape(150, 2, _V, 10)[:, 0])
    G = jnp.concatenate(taps, axis=2)
    z = _dgb(G.reshape(150 * _V, 120), w2_ref[...], ((1,), (0,)))
    cp = jnp.pad(yr[:, :, 40:50], ((1, 1), (0, 0), (0, 0)),
                 constant_values=-1e30)
    mp = jnp.maximum(jnp.maximum(cp[0:_T], cp[1:_T + 1]), cp[2:_T + 2])
    mp = mp.reshape(150, 2, _V, 10)[:, 0].reshape(150 * _V, 10)
    b5 = y3[:, :, 50:60].reshape(150, 2, _V, 10)[:, 0].reshape(150 * _V, 10)
    xs = x3.reshape(150, 2, _V, _C)[:, 0].reshape(150 * _V, _C)
    r = _dgb(xs, wr_ref[...], ((1,), (1,)))
    out = jnp.concatenate([z, mp, b5], axis=1) + r
    o_ref[0] = jnp.maximum(out, 0.0).reshape(150, _V, _C)


def _make_tcn_s1_body(act, prologue):
    T2 = 150

    def body(*refs):
        if prologue:
            a_ref, b_ref, c_ref, w1_ref, w2_ref, o_ref = refs
            x3 = jnp.maximum(a_ref[0] + b_ref[0] + c_ref[0], 0.0)
        else:
            x_ref, w1_ref, w2_ref, o_ref = refs
            x3 = x_ref[0]
        x2 = x3.reshape(T2 * _V, _C)
        y = _dgb(x2, w1_ref[...], ((1,), (1,)))
        y3 = y.reshape(T2, _V, _C)
        yr = jnp.maximum(y3[:, :, :50], 0.0)
        yp = jnp.pad(yr[:, :, :40], ((4, 4), (0, 0), (0, 0)))
        taps = []
        for j in range(3):
            for i in range(4):
                d = i + 1
                off = 4 + (j - 1) * d
                taps.append(yp[off:off + T2, :, 10 * i:10 * i + 10])
        G = jnp.concatenate(taps, axis=2)
        z = _dgb(G.reshape(T2 * _V, 120), w2_ref[...], ((1,), (0,)))
        cp = jnp.pad(yr[:, :, 40:50], ((1, 1), (0, 0), (0, 0)),
                     constant_values=-1e30)
        mp = jnp.maximum(jnp.maximum(cp[0:T2], cp[1:T2 + 1]), cp[2:T2 + 2])
        mp = mp.reshape(T2 * _V, 10)
        b5 = y3[:, :, 50:60].reshape(T2 * _V, 10)
        out = jnp.concatenate([z, mp, b5], axis=1) + x2
        if act:
            out = jnp.maximum(out, 0.0)
        o_ref[0] = out.reshape(T2, _V, _C)

    return body


def _make_g3d_body(window):
    pad = (window - 1) // 2
    T2 = 150

    def body(x_ref, dall_ref, wg_ref, wp_ref, wo_ref, o_ref):
        x3 = x_ref[0]
        xp = jnp.pad(x3, ((pad, pad), (0, 0), (0, 0)))
        cols = []
        for j in range(window):
            cols.append(xp[j:j + _T].reshape(T2, 2, _V, _C)[:, 0])
        P = cols[0]
        for c in cols[1:]:
            P = P + c
        Qs = _dgb(P, dall_ref[...], ((1,), (0,)))
        QW = None
        for kk in range(_KD - 1):
            qk = Qs[:, :, kk * _V:(kk + 1) * _V]
            wgk = wg_ref[...][:, (kk + 1) * _C:(kk + 2) * _C]
            zqk = _dgb(qk, wgk, ((1,), (1,)))
            QW = zqk if QW is None else QW + zqk
        xw4 = jnp.concatenate([c[:, None] for c in cols], axis=1)
        hs = []
        for v in range(_V):
            xv = xw4[:, :, v, :]
            zv = _dgb(xv, wp_ref[v], ((2,), (1,)))
            hv = jnp.maximum(zv + QW[:, v, :][:, None, :], 0.0)
            hs.append(hv[:, :, None, :])
        h4 = jnp.concatenate(hs, axis=2)
        out = None
        for j in range(window):
            hj = h4[:, j].reshape(T2 * _V, _C)
            oj = _dotb(hj, wo_ref[j])
            out = oj if out is None else out + oj
        o_ref[0] = out.reshape(T2, _V, _C)

    return body


def _pool_body(x_ref, w_ref, b_ref, o_ref):
    x4 = x_ref[...].reshape(8, _V, _C, 150)
    p = jnp.sum(x4, axis=(1, 3))
    pr = _dg(p, w_ref[...], ((1,), (0,)))
    logits = pr * (1.0 / 3750.0) + b_ref[...]
    m = jnp.max(logits, axis=1, keepdims=True)
    zz = logits - m
    lse = jnp.log(jnp.sum(jnp.exp(zz), axis=1, keepdims=True))
    o_ref[...] = zz - lse


def _full(shape):
    nd = len(shape)
    return pl.BlockSpec(shape, lambda n, *, _nd=nd: (0,) * _nd)


def _per_n(shape_tail):
    nd = len(shape_tail)
    return pl.BlockSpec((1,) + shape_tail,
                        lambda n, *, _nd=nd: (n,) + (0,) * _nd)


def _call(body, grid_n, in_arrays, in_specs, out_shape, out_spec):
    return pl.pallas_call(
        body,
        grid=(grid_n,),
        in_specs=in_specs,
        out_specs=out_spec,
        out_shape=jax.ShapeDtypeStruct(out_shape, _F32),
        compiler_params=pltpu.CompilerParams(
            dimension_semantics=("parallel",),
            vmem_limit_bytes=100 * 1024 * 1024),
    )(*in_arrays)


def _w2cat(w2):
    eye = jnp.eye(4, dtype=_F32)
    return jnp.einsum('ionj,ik->jinko', w2, eye).reshape(120, 40)


def kernel(x, W_gcn1, tcn_a_w1, tcn_a_w2, tcn_a_res, tcn_b_w1, tcn_b_w2,
           g3d_w3_gcn, g3d_w3_out, g3d_w5_gcn, g3d_w5_out, tcn3_w1, tcn3_w2,
           fc_w, fc_b):
    VC = _V * _C
    x2d = x.reshape(_N, _T, VC)
    M2 = jnp.einsum('kvu,okc->ucvo', _A1S,
                    W_gcn1.reshape(_C, _KG, _C)).reshape(VC, VC)
    h1 = _call(_gcn1_body, _N, (x2d, M2),
               [_per_n((_T, VC)), _full((VC, VC))],
               (_N, _T, VC), _per_n((_T, VC)))
    h1v = h1.reshape(_N, _T, _V, _C)

    ha = _call(_tcn_a_body, _N,
               (h1v, tcn_a_w1.reshape(_C, _C), _w2cat(tcn_a_w2), tcn_a_res),
               [_per_n((_T, _V, _C)), _full((_C, _C)), _full((120, 40)),
                _full((_C, _C))],
               (_N, 150, _V, _C), _per_n((150, _V, _C)))

    hb = _call(_make_tcn_s1_body(False, False), _N,
               (ha, tcn_b_w1.reshape(_C, _C), _w2cat(tcn_b_w2)),
               [_per_n((150, _V, _C)), _full((_C, _C)), _full((120, 40))],
               (_N, 150, _V, _C), _per_n((150, _V, _C)))

    outs_g = []
    for window, dall, beta, wg, wo in (
            (3, _D3, _BETA3, g3d_w3_gcn, g3d_w3_out),
            (5, _D5, _BETA5, g3d_w5_gcn, g3d_w5_out)):
        wp = jnp.einsum('kv,ikc->vic', beta, wg.reshape(_C, _KD, _C))
        woT = jnp.transpose(wo, (2, 1, 0))
        g = _call(_make_g3d_body(window), _N,
                  (x, dall, wg, wp, woT),
                  [_per_n((_T, _V, _C)), _full((_V, (_KD - 1) * _V)),
                   _full((_C, _KD * _C)), _full((_V, _C, _C)),
                   _full((window, _C, _C))],
                  (_N, 150, _V, _C), _per_n((150, _V, _C)))
        outs_g.append(g)

    h2 = _call(_make_tcn_s1_body(True, True), _N,
               (hb, outs_g[0], outs_g[1], tcn3_w1.reshape(_C, _C),
                _w2cat(tcn3_w2)),
               [_per_n((150, _V, _C)), _per_n((150, _V, _C)),
                _per_n((150, _V, _C)), _full((_C, _C)), _full((120, 40))],
               (_N, 150, _V, _C), _per_n((150, _V, _C)))

    h2t = jnp.transpose(h2, (0, 3, 1, 2)).reshape(_N, VC, 150)
    out = _call(_pool_body, _N // 8,
                (h2t, fc_w.T, fc_b.reshape(1, _C)),
                [pl.BlockSpec((8, VC, 150), lambda n: (n, 0, 0)),
                 _full((_C, _C)), _full((1, _C))],
                (_N, _C), pl.BlockSpec((8, _C), lambda n: (n, 0)))
    return out
